# baseline (speedup 1.0000x reference)
import functools

import jax
import jax.numpy as jnp
from jax import lax
from jax.experimental import pallas as pl
from jax.experimental.pallas import tpu as pltpu
from jax.experimental.pallas import tpu_sc as plsc

N = 10000
E = 160000
D = 256
HALFD = 128
NT = 16
EPT = E // NT
CK = 128
NCH = 80
EPAD = NCH * CK
NPAD = N + 8
GARB = N
RPT = N // NT

_mesh = plsc.VectorSubcoreMesh(core_axis_name="c", subcore_axis_name="s",
                               num_cores=2, num_subcores=NT)


HPAD = EPAD


@functools.partial(
    pl.kernel,
    out_type=jax.ShapeDtypeStruct((N,), jnp.float32),
    mesh=_mesh,
    scratch_types=[
        pltpu.VMEM((NCH, CK), jnp.int32),
        pltpu.VMEM((CK,), jnp.float32),
        pltpu.VMEM((HPAD,), jnp.float32),
        pltpu.VMEM((640,), jnp.float32),
        pltpu.VMEM((640,), jnp.float32),
        pltpu.VMEM_SHARED((NT * HPAD,), jnp.float32),
    ],
)
def _sc_degree(dstpo_hbm, zeros_hbm, ones_hbm, deg_hbm, dst2d, ones1, zbuf, tmp, accv, hists):
    c = lax.axis_index("c")
    s = lax.axis_index("s")
    pltpu.sync_copy(dstpo_hbm.at[s], dst2d)
    pltpu.sync_copy(ones_hbm, ones1)
    pltpu.sync_copy(zeros_hbm, zbuf)
    pltpu.sync_copy(zbuf, hists.at[pl.ds(s * HPAD, HPAD)])

    @pl.loop(0, NCH)
    def _(j):
        pltpu.sync_copy(ones1, hists.at[dst2d.at[j]], add=True)

    plsc.subcore_barrier()
    def _reduce(off, ln, wr):
        for o in range(0, ln, 16):
            accv[pl.ds(o, 16)] = jnp.zeros((16,), jnp.float32)
        for k in range(NT):
            pltpu.sync_copy(hists.at[pl.ds(k * HPAD + off, ln)], tmp.at[pl.ds(0, ln)])
            for o in range(0, ln, 16):
                accv[pl.ds(o, 16)] = accv[pl.ds(o, 16)] + tmp[pl.ds(o, 16)]
        @pl.when(c == 0)
        def _():
            pltpu.sync_copy(accv.at[pl.ds(0, wr)], deg_hbm.at[pl.ds(off, wr)])

    @pl.when(s < 15)
    def _():
        _reduce(s * 640, 640, 640)
    @pl.when(s == 15)
    def _():
        _reduce(9600, 512, 400)


@functools.partial(
    pl.kernel,
    out_type=jax.ShapeDtypeStruct((2, N, HALFD), jnp.float32),
    mesh=_mesh,
    scratch_types=[
        pltpu.VMEM((NCH, CK), jnp.int32),
        pltpu.VMEM((2, CK), jnp.int32),
        pltpu.VMEM((CK, HALFD), jnp.float32),
        pltpu.VMEM((CK, HALFD), jnp.float32),
        pltpu.VMEM_SHARED((NPAD, HALFD), jnp.float32),
        pltpu.SemaphoreType.DMA,
        pltpu.SemaphoreType.DMA,
        pltpu.SemaphoreType.DMA,
    ],
)
def _sc_aggregate(hs_hbm, srcp_hbm, dstp_hbm, out_hbm, src2d, dring,
                  rows0, rows1, acc, sem0, sem1, isem):
    c = lax.axis_index("c")
    s = lax.axis_index("s")
    @pl.when(s < 15)
    def _():
        r0 = s * 624
        pltpu.async_copy(hs_hbm.at[c].at[pl.ds(r0, 624)], acc.at[pl.ds(r0, 624)], isem)
    @pl.when(s == 15)
    def _():
        pltpu.async_copy(hs_hbm.at[c].at[pl.ds(9360, 640)], acc.at[pl.ds(9360, 640)], isem)
    pltpu.sync_copy(srcp_hbm.at[s], src2d)
    @pl.when(s < 15)
    def _():
        r0 = s * 624
        pltpu.make_async_copy(hs_hbm.at[c].at[pl.ds(r0, 624)], acc.at[pl.ds(r0, 624)], isem).wait()
    @pl.when(s == 15)
    def _():
        pltpu.make_async_copy(hs_hbm.at[c].at[pl.ds(9360, 640)], acc.at[pl.ds(9360, 640)], isem).wait()
    plsc.subcore_barrier()

    bufs = ((rows0, sem0), (rows1, sem1))
    for b in range(2):
        rows, sem = bufs[b]
        pltpu.async_copy(hs_hbm.at[c].at[src2d.at[b]], rows, sem)
        pltpu.async_copy(dstp_hbm.at[s].at[b], dring.at[b], sem)

    @pl.loop(0, NCH // 2)
    def _(g):
        j = g * 2
        for b in range(2):
            rows, sem = bufs[b]
            jb = j + b
            pltpu.make_async_copy(hs_hbm.at[c].at[src2d.at[jb]], rows, sem).wait()
            pltpu.make_async_copy(dstp_hbm.at[s].at[jb], dring.at[b], sem).wait()
            pltpu.sync_copy(rows, acc.at[dring.at[b]], add=True)
            @pl.when(jb + 2 < NCH)
            def _():
                pltpu.async_copy(hs_hbm.at[c].at[src2d.at[jb + 2]], rows, sem)
                pltpu.async_copy(dstp_hbm.at[s].at[jb + 2], dring.at[b], sem)

    plsc.subcore_barrier()
    @pl.when(s < 15)
    def _():
        r0 = s * 624
        pltpu.sync_copy(acc.at[pl.ds(r0, 624)], out_hbm.at[c].at[pl.ds(r0, 624)])
    @pl.when(s == 15)
    def _():
        pltpu.sync_copy(acc.at[pl.ds(9360, 640)], out_hbm.at[c].at[pl.ds(9360, 640)])


_RB = 1000


def _mm1_body(x_ref, w_ref, deg_ref, out_ref):
    dis = lax.rsqrt(deg_ref[...] + 1.0)
    h = jnp.dot(x_ref[...], w_ref[...], preferred_element_type=jnp.float32) * dis
    out_ref[...] = jnp.stack([h[:, :HALFD], h[:, HALFD:]])


def _mm2_body(agg_ref, deg_ref, b_ref, w_ref, out_ref):
    dis = lax.rsqrt(deg_ref[...] + 1.0)
    a = jnp.concatenate([agg_ref[0], agg_ref[1]], axis=-1)
    h1 = jnp.maximum(a * dis + b_ref[...], 0.0)
    h2 = jnp.dot(h1, w_ref[...], preferred_element_type=jnp.float32) * dis
    out_ref[...] = jnp.stack([h2[:, :HALFD], h2[:, HALFD:]])


def _final_body(x_ref, agg_ref, deg_ref, b_ref, out_ref):
    dis = lax.rsqrt(deg_ref[...] + 1.0)
    a = jnp.concatenate([agg_ref[0], agg_ref[1]], axis=-1)
    h = jnp.maximum(a * dis + b_ref[...], 0.0)
    out_ref[...] = (x_ref[...] + h) * 0.5


_x_spec = pl.BlockSpec((_RB, D), lambda i: (i, 0))
_w_spec = pl.BlockSpec((D, D), lambda i: (0, 0))
_deg_spec = pl.BlockSpec((_RB, 1), lambda i: (i, 0))
_b_spec = pl.BlockSpec((1, D), lambda i: (0, 0))
_split_spec = pl.BlockSpec((2, _RB, HALFD), lambda i: (0, i, 0))

_mm1 = pl.pallas_call(
    _mm1_body,
    grid=(N // _RB,),
    in_specs=[_x_spec, _w_spec, _deg_spec],
    out_specs=_split_spec,
    out_shape=jax.ShapeDtypeStruct((2, N, HALFD), jnp.float32),
)

_mm2 = pl.pallas_call(
    _mm2_body,
    grid=(N // _RB,),
    in_specs=[_split_spec, _deg_spec, _b_spec, _w_spec],
    out_specs=_split_spec,
    out_shape=jax.ShapeDtypeStruct((2, N, HALFD), jnp.float32),
)

_final = pl.pallas_call(
    _final_body,
    grid=(N // _RB,),
    in_specs=[_x_spec, _split_spec, _deg_spec, _b_spec],
    out_specs=_x_spec,
    out_shape=jax.ShapeDtypeStruct((N, D), jnp.float32),
)


@jax.jit
def kernel(x, edge_index, W1, b1, W2, b2):
    src = edge_index[0].astype(jnp.int32)
    dst = edge_index[1].astype(jnp.int32)
    pad = EPAD - EPT
    srcp = jnp.pad(src.reshape(NT, EPT), ((0, 0), (0, pad)),
                   constant_values=0).reshape(NT, NCH, CK)
    dstp = jnp.pad(dst.reshape(NT, EPT), ((0, 0), (0, pad)),
                   constant_values=GARB).reshape(NT, NCH, CK)
    zeros1 = jnp.zeros((HPAD,), jnp.float32)
    ones1 = jnp.ones((CK,), jnp.float32)
    dstpo = dstp + (jnp.arange(NT, dtype=jnp.int32) * HPAD)[:, None, None]

    deg = _sc_degree(dstpo, zeros1, ones1).reshape(N, 1)
    hs1 = _mm1(x, W1, deg)
    agg1 = _sc_aggregate(hs1, srcp, dstp)
    hs2 = _mm2(agg1, deg, b1.reshape(1, D), W2)
    agg2 = _sc_aggregate(hs2, srcp, dstp)
    return _final(x, agg2, deg, b2.reshape(1, D))


# speedup vs baseline: 1.0504x; 1.0504x over previous
import functools

import jax
import jax.numpy as jnp
from jax import lax
from jax.experimental import pallas as pl
from jax.experimental.pallas import tpu as pltpu
from jax.experimental.pallas import tpu_sc as plsc

N = 10000
E = 160000
D = 256
HALFD = 128
NT = 16
EPT = E // NT
CK = 128
NCH = (EPT + CK - 1) // CK
EPAD = NCH * CK
NPAD = N + 8
GARB = N
RPT = N // NT

_mesh = plsc.VectorSubcoreMesh(core_axis_name="c", subcore_axis_name="s",
                               num_cores=2, num_subcores=NT)


HPAD = EPAD
NCHD = 40


@functools.partial(
    pl.kernel,
    out_type=jax.ShapeDtypeStruct((2 * N,), jnp.float32),
    mesh=_mesh,
    scratch_types=[
        pltpu.VMEM((NCHD, CK), jnp.int32),
        pltpu.VMEM((CK,), jnp.float32),
        pltpu.VMEM((HPAD,), jnp.float32),
        pltpu.VMEM((640,), jnp.float32),
        pltpu.VMEM((640,), jnp.float32),
        pltpu.VMEM_SHARED((NT * HPAD,), jnp.float32),
    ],
)
def _sc_degree(dstpo_hbm, zeros_hbm, ones_hbm, deg_hbm, dst2d, ones1, zbuf, tmp, accv, hists):
    c = lax.axis_index("c")
    s = lax.axis_index("s")
    pltpu.sync_copy(dstpo_hbm.at[c].at[s], dst2d)
    pltpu.sync_copy(ones_hbm, ones1)
    pltpu.sync_copy(zeros_hbm, zbuf)
    pltpu.sync_copy(zbuf, hists.at[pl.ds(s * HPAD, HPAD)])

    @pl.loop(0, NCHD)
    def _(j):
        pltpu.sync_copy(ones1, hists.at[dst2d.at[j]], add=True)

    plsc.subcore_barrier()
    def _reduce(off, ln, wr):
        for o in range(0, ln, 16):
            accv[pl.ds(o, 16)] = jnp.zeros((16,), jnp.float32)
        for k in range(NT):
            pltpu.sync_copy(hists.at[pl.ds(k * HPAD + off, ln)], tmp.at[pl.ds(0, ln)])
            for o in range(0, ln, 16):
                accv[pl.ds(o, 16)] = accv[pl.ds(o, 16)] + tmp[pl.ds(o, 16)]
        pltpu.sync_copy(accv.at[pl.ds(0, wr)], deg_hbm.at[pl.ds(c * N + off, wr)])

    @pl.when(s < 15)
    def _():
        _reduce(s * 640, 640, 640)
    @pl.when(s == 15)
    def _():
        _reduce(9600, 512, 400)


@functools.partial(
    pl.kernel,
    out_type=jax.ShapeDtypeStruct((2, N, HALFD), jnp.float32),
    mesh=_mesh,
    scratch_types=[
        pltpu.VMEM((NCH, CK), jnp.int32),
        pltpu.VMEM((NCH, CK), jnp.int32),
        pltpu.VMEM((CK, HALFD), jnp.float32),
        pltpu.VMEM_SHARED((NPAD, HALFD), jnp.float32),
        pltpu.SemaphoreType.DMA,
    ],
)
def _sc_aggregate(hs_hbm, srcp_hbm, dstp_hbm, out_hbm, src2d, dst2d, rows, acc, sem):
    c = lax.axis_index("c")
    s = lax.axis_index("s")
    pltpu.sync_copy(srcp_hbm.at[s], src2d)
    pltpu.sync_copy(dstp_hbm.at[s], dst2d)
    @pl.when(s < 15)
    def _():
        r0 = s * 624
        pltpu.sync_copy(hs_hbm.at[c].at[pl.ds(r0, 624)], acc.at[pl.ds(r0, 624)])
    @pl.when(s == 15)
    def _():
        pltpu.sync_copy(hs_hbm.at[c].at[pl.ds(9360, 640)], acc.at[pl.ds(9360, 640)])
    plsc.subcore_barrier()

    @pl.loop(0, NCH)
    def _(j):
        pltpu.async_copy(hs_hbm.at[c].at[src2d.at[j]], rows, sem).wait()
        pltpu.sync_copy(rows, acc.at[dst2d.at[j]], add=True)

    plsc.subcore_barrier()
    @pl.when(s < 15)
    def _():
        r0 = s * 624
        pltpu.sync_copy(acc.at[pl.ds(r0, 624)], out_hbm.at[c].at[pl.ds(r0, 624)])
    @pl.when(s == 15)
    def _():
        pltpu.sync_copy(acc.at[pl.ds(9360, 640)], out_hbm.at[c].at[pl.ds(9360, 640)])


_RB = 2000


def _dis(deg_ref):
    return lax.rsqrt(deg_ref[...] + 1.0)


def _mm1_body(x_ref, w_ref, deg_ref, out_ref):
    h = jnp.dot(x_ref[...], w_ref[...], preferred_element_type=jnp.float32) * _dis(deg_ref)
    out_ref[...] = jnp.stack([h[:, :HALFD], h[:, HALFD:]])


def _mm2_body(agg_ref, deg_ref, b_ref, w_ref, out_ref):
    dis = _dis(deg_ref)
    a = jnp.concatenate([agg_ref[0], agg_ref[1]], axis=-1)
    h1 = jnp.maximum(a * dis + b_ref[...], 0.0)
    h2 = jnp.dot(h1, w_ref[...], preferred_element_type=jnp.float32) * dis
    out_ref[...] = jnp.stack([h2[:, :HALFD], h2[:, HALFD:]])


def _final_body(x_ref, agg_ref, deg_ref, b_ref, out_ref):
    a = jnp.concatenate([agg_ref[0], agg_ref[1]], axis=-1)
    h = jnp.maximum(a * _dis(deg_ref) + b_ref[...], 0.0)
    out_ref[...] = (x_ref[...] + h) * 0.5


_x_spec = pl.BlockSpec((_RB, D), lambda i: (i, 0))
_w_spec = pl.BlockSpec((D, D), lambda i: (0, 0))
_deg_spec = pl.BlockSpec((_RB, 1), lambda i: (i, 0))
_b_spec = pl.BlockSpec((1, D), lambda i: (0, 0))
_split_spec = pl.BlockSpec((2, _RB, HALFD), lambda i: (0, i, 0))

_mm1 = pl.pallas_call(
    _mm1_body,
    grid=(N // _RB,),
    in_specs=[_x_spec, _w_spec, _deg_spec],
    out_specs=_split_spec,
    out_shape=jax.ShapeDtypeStruct((2, N, HALFD), jnp.float32),
)

_mm2 = pl.pallas_call(
    _mm2_body,
    grid=(N // _RB,),
    in_specs=[_split_spec, _deg_spec, _b_spec, _w_spec],
    out_specs=_split_spec,
    out_shape=jax.ShapeDtypeStruct((2, N, HALFD), jnp.float32),
)

_final = pl.pallas_call(
    _final_body,
    grid=(N // _RB,),
    in_specs=[_x_spec, _split_spec, _deg_spec, _b_spec],
    out_specs=_x_spec,
    out_shape=jax.ShapeDtypeStruct((N, D), jnp.float32),
)


@jax.jit
def kernel(x, edge_index, W1, b1, W2, b2):
    src = edge_index[0].astype(jnp.int32)
    dst = edge_index[1].astype(jnp.int32)
    pad = EPAD - EPT
    srcp = jnp.pad(src.reshape(NT, EPT), ((0, 0), (0, pad)),
                   constant_values=0).reshape(NT, NCH, CK)
    dstp = jnp.pad(dst.reshape(NT, EPT), ((0, 0), (0, pad)),
                   constant_values=GARB).reshape(NT, NCH, CK)
    zeros1 = jnp.zeros((HPAD,), jnp.float32)
    ones1 = jnp.ones((CK,), jnp.float32)
    dpad = NCHD * CK - E // 32
    dstc = jnp.pad(dst.reshape(2, NT, E // 32), ((0, 0), (0, 0), (0, dpad)),
                   constant_values=GARB).reshape(2, NT, NCHD, CK)
    dstpo = dstc + (jnp.arange(NT, dtype=jnp.int32) * HPAD)[None, :, None, None]

    degp = _sc_degree(dstpo, zeros1, ones1).reshape(2, N)
    deg = (degp[0] + degp[1]).reshape(N, 1)
    hs1 = _mm1(x, W1, deg)
    agg1 = _sc_aggregate(hs1, srcp, dstp)
    hs2 = _mm2(agg1, deg, b1.reshape(1, D), W2)
    agg2 = _sc_aggregate(hs2, srcp, dstp)
    return _final(x, agg2, deg, b2.reshape(1, D))
